# baseline (speedup 1.0000x reference)
import jax
import jax.numpy as jnp
from jax.experimental import pallas as pl
from jax.experimental.pallas import tpu as pltpu

_L = 24
_LPAD = 32
_F = 1024


def _gnn_body(xn_hbm, adj_hbm, ws_ref, wn_ref, b_ref, uv_hbm,
              xn_ref, adj_ref, uv_s, xn_sems, adj_sem, out_sems):
    pltpu.make_async_copy(adj_hbm, adj_ref, adj_sem).start()
    lo = pl.ds(0, _F // 2)
    hi = pl.ds(_F // 2, _F // 2)
    pltpu.make_async_copy(xn_hbm.at[:, lo], xn_ref.at[:, lo],
                          xn_sems.at[0]).start()
    pltpu.make_async_copy(xn_hbm.at[:, hi], xn_ref.at[:, hi],
                          xn_sems.at[1]).start()

    def bd(W):
        Wp = jnp.pad(W, ((0, 0), (0, _LPAD - _L), (0, _LPAD - _L)))
        z = jnp.zeros_like(Wp)
        rows = [jnp.concatenate([Wp if c == r else z for c in range(4)], axis=2)
                for r in range(4)]
        return jnp.concatenate(rows, axis=1)

    WsB = bd(ws_ref[...])
    WnB = bd(wn_ref[...])
    bB = jnp.tile(jnp.pad(b_ref[...], ((0, 0), (0, _LPAD - _L))), (1, 32))

    pltpu.make_async_copy(adj_hbm, adj_ref, adj_sem).wait()
    A = adj_ref[...]
    Ab = (A != 0).astype(jnp.float32)
    deg = jnp.sum(Ab, axis=0)
    deg_inv = jnp.where(deg > 0, 1.0 / jnp.maximum(deg, 1.0), 0.0)
    A_s = Ab * deg_inv[None, :]

    dnT = (((0,), (0,)), ((), ()))
    dnW = (((1,), (1,)), ((), ()))

    def aggT(H):
        return jax.lax.dot_general(A_s, H, dnT,
                                   preferred_element_type=jnp.float32)

    def lin(H, W2):
        nt = H.shape[1] // 128
        cols = [
            jax.lax.dot_general(H[:, 128 * t:128 * (t + 1)], W2, dnW,
                                preferred_element_type=jnp.float32)
            for t in range(nt)
        ]
        return jnp.concatenate(cols, axis=1)

    pltpu.make_async_copy(xn_hbm.at[:, lo], xn_ref.at[:, lo],
                          xn_sems.at[0]).wait()
    Tl = aggT(xn_ref[:, lo])
    pltpu.make_async_copy(xn_hbm.at[:, hi], xn_ref.at[:, hi],
                          xn_sems.at[1]).wait()
    Th = aggT(xn_ref[:, hi])
    T = jnp.concatenate([Tl, Th], axis=1)
    Xn = xn_ref[...]
    V0 = deg[:, None] * T
    AV = aggT(V0)
    U1 = lin(4.0 * Xn, WsB[0]) + lin(4.0 * T, WnB[0]) + bB[0][None, :]
    V1 = lin(V0, WsB[0]) + lin(AV, WnB[0]) + bB[0][None, :]

    UV = jnp.concatenate([U1, V1], axis=1)
    b2 = jnp.concatenate([bB[1], bB[1]])
    AUV = aggT(UV)
    UV = lin(UV, WsB[1]) + lin(AUV, WnB[1]) + b2[None, :]

    AUV = aggT(UV)
    Uo = lin(UV[:, :_F], WsB[2]) + lin(AUV[:, :_F], WnB[2]) + bB[2][None, :]
    uv_s[:, :_F] = Uo
    pltpu.make_async_copy(uv_s.at[:, pl.ds(0, _F)],
                          uv_hbm.at[:, pl.ds(0, _F)], out_sems.at[0]).start()
    Vo = lin(UV[:, _F:], WsB[2]) + lin(AUV[:, _F:], WnB[2]) + bB[2][None, :]
    uv_s[:, _F:] = Vo
    pltpu.make_async_copy(uv_s.at[:, pl.ds(_F, _F)],
                          uv_hbm.at[:, pl.ds(_F, _F)], out_sems.at[1]).start()
    pltpu.make_async_copy(uv_s.at[:, pl.ds(0, _F)],
                          uv_hbm.at[:, pl.ds(0, _F)], out_sems.at[0]).wait()
    pltpu.make_async_copy(uv_s.at[:, pl.ds(_F, _F)],
                          uv_hbm.at[:, pl.ds(_F, _F)], out_sems.at[1]).wait()


def kernel(x, adj, W_self, b_self, W_neigh):
    nS, nC, nN, L = x.shape
    nG = nC * nS

    Xn = jnp.transpose(x, (2, 0, 1, 3))
    Xn = jnp.pad(Xn, ((0, 0), (0, 0), (0, 0), (0, _LPAD - L)))
    Xn = Xn.reshape(nN, nG * _LPAD)

    UV = pl.pallas_call(
        _gnn_body,
        in_specs=[
            pl.BlockSpec(memory_space=pl.ANY),
            pl.BlockSpec(memory_space=pl.ANY),
            pl.BlockSpec(memory_space=pltpu.VMEM),
            pl.BlockSpec(memory_space=pltpu.VMEM),
            pl.BlockSpec(memory_space=pltpu.VMEM),
        ],
        out_specs=pl.BlockSpec(memory_space=pl.ANY),
        out_shape=jax.ShapeDtypeStruct((nN, 2 * _F), jnp.float32),
        scratch_shapes=[
            pltpu.VMEM((nN, _F), jnp.float32),
            pltpu.VMEM((nN, nN), jnp.float32),
            pltpu.VMEM((nN, 2 * _F), jnp.float32),
            pltpu.SemaphoreType.DMA((2,)),
            pltpu.SemaphoreType.DMA,
            pltpu.SemaphoreType.DMA((2,)),
        ],
    )(Xn, adj, W_self, W_neigh, b_self)

    Ur = UV[:, :_F].reshape(nN, nS, nC, _LPAD)[..., :L].transpose(1, 2, 0, 3)
    Vr = UV[:, _F:].reshape(nN, nS, nC, _LPAD)[..., :L].transpose(1, 2, 0, 3)
    out = jnp.stack([Ur, Vr], axis=2).reshape(nS, 2 * nC, nN, L)
    return out


# speedup vs baseline: 1.4107x; 1.4107x over previous
import jax
import jax.numpy as jnp
from jax.experimental import pallas as pl
from jax.experimental.pallas import tpu as pltpu

_NLAYER = 3
_L = 24
_LPAD = 32
_NTILE = 8


def _gnn_body(xn_hbm, adj_hbm, ws_ref, wn_ref, b_ref, u_hbm, v_hbm,
              xn_ref, adj_ref, u_s, v_s, xn_sem, adj_sem, out_sems):
    pltpu.make_async_copy(adj_hbm, adj_ref, adj_sem).start()
    pltpu.make_async_copy(xn_hbm, xn_ref, xn_sem).start()

    def bd(W):
        Wp = jnp.pad(W, ((0, 0), (0, _LPAD - _L), (0, _LPAD - _L)))
        z = jnp.zeros_like(Wp)
        rows = [jnp.concatenate([Wp if c == r else z for c in range(4)], axis=2)
                for r in range(4)]
        return jnp.concatenate(rows, axis=1)

    WsB = bd(ws_ref[...])
    WnB = bd(wn_ref[...])
    bB = jnp.tile(jnp.pad(b_ref[...], ((0, 0), (0, _LPAD - _L))), (1, 32))

    pltpu.make_async_copy(adj_hbm, adj_ref, adj_sem).wait()
    A = adj_ref[...]
    Ab = (A != 0).astype(jnp.float32)
    deg = jnp.sum(Ab, axis=0)
    deg_inv = jnp.where(deg > 0, 1.0 / jnp.maximum(deg, 1.0), 0.0)
    A_s = Ab * deg_inv[None, :]

    dnT = (((0,), (0,)), ((), ()))
    dnW = (((1,), (1,)), ((), ()))

    def aggT(H):
        return jax.lax.dot_general(A_s, H, dnT,
                                   preferred_element_type=jnp.float32)

    def lin(H, W2):
        cols = [
            jax.lax.dot_general(H[:, 128 * t:128 * (t + 1)], W2, dnW,
                                preferred_element_type=jnp.float32)
            for t in range(_NTILE)
        ]
        return jnp.concatenate(cols, axis=1)

    pltpu.make_async_copy(xn_hbm, xn_ref, xn_sem).wait()
    Xn = xn_ref[...].astype(jnp.float32)

    T = aggT(Xn)
    U = 4.0 * Xn
    V = deg[:, None] * T
    AV = aggT(V)
    U = lin(U, WsB[0]) + lin(4.0 * T, WnB[0]) + bB[0][None, :]
    V = lin(V, WsB[0]) + lin(AV, WnB[0]) + bB[0][None, :]
    for i in range(1, _NLAYER):
        AU = aggT(U)
        AV = aggT(V)
        U = lin(U, WsB[i]) + lin(AU, WnB[i]) + bB[i][None, :]
        V = lin(V, WsB[i]) + lin(AV, WnB[i]) + bB[i][None, :]

    u_s[...] = U.astype(jnp.bfloat16)
    pltpu.make_async_copy(u_s, u_hbm, out_sems.at[0]).start()
    v_s[...] = V.astype(jnp.bfloat16)
    pltpu.make_async_copy(v_s, v_hbm, out_sems.at[1]).start()
    pltpu.make_async_copy(u_s, u_hbm, out_sems.at[0]).wait()
    pltpu.make_async_copy(v_s, v_hbm, out_sems.at[1]).wait()


def kernel(x, adj, W_self, b_self, W_neigh):
    nS, nC, nN, L = x.shape
    nG = nC * nS

    Xn = jnp.transpose(x, (2, 0, 1, 3))
    Xn = jnp.pad(Xn, ((0, 0), (0, 0), (0, 0), (0, _LPAD - L)))
    Xn = Xn.reshape(nN, nG * _LPAD).astype(jnp.bfloat16)

    U, V = pl.pallas_call(
        _gnn_body,
        in_specs=[
            pl.BlockSpec(memory_space=pl.ANY),
            pl.BlockSpec(memory_space=pl.ANY),
            pl.BlockSpec(memory_space=pltpu.VMEM),
            pl.BlockSpec(memory_space=pltpu.VMEM),
            pl.BlockSpec(memory_space=pltpu.VMEM),
        ],
        out_specs=[
            pl.BlockSpec(memory_space=pl.ANY),
            pl.BlockSpec(memory_space=pl.ANY),
        ],
        out_shape=[
            jax.ShapeDtypeStruct((nN, nG * _LPAD), jnp.bfloat16),
            jax.ShapeDtypeStruct((nN, nG * _LPAD), jnp.bfloat16),
        ],
        scratch_shapes=[
            pltpu.VMEM((nN, nG * _LPAD), jnp.bfloat16),
            pltpu.VMEM((nN, nN), jnp.float32),
            pltpu.VMEM((nN, nG * _LPAD), jnp.bfloat16),
            pltpu.VMEM((nN, nG * _LPAD), jnp.bfloat16),
            pltpu.SemaphoreType.DMA,
            pltpu.SemaphoreType.DMA,
            pltpu.SemaphoreType.DMA((2,)),
        ],
    )(Xn, adj, W_self, W_neigh, b_self)

    U = U.astype(jnp.float32)
    V = V.astype(jnp.float32)
    Ur = U.reshape(nN, nS, nC, _LPAD)[..., :L].transpose(1, 2, 0, 3)
    Vr = V.reshape(nN, nS, nC, _LPAD)[..., :L].transpose(1, 2, 0, 3)
    out = jnp.stack([Ur, Vr], axis=2).reshape(nS, 2 * nC, nN, L)
    return out


# speedup vs baseline: 1.4167x; 1.0042x over previous
import jax
import jax.numpy as jnp
from jax.experimental import pallas as pl
from jax.experimental.pallas import tpu as pltpu

_NLAYER = 3
_L = 24
_LPAD = 32
_NTILE = 8


def _gnn_body(xn_hbm, adj_hbm, ws_ref, wn_ref, b_ref, u_hbm, v_hbm,
              xn_ref, adj_ref, u_s, v_s, xn_sem, adj_sem, out_sems):
    pltpu.make_async_copy(adj_hbm, adj_ref, adj_sem).start()
    pltpu.make_async_copy(xn_hbm, xn_ref, xn_sem).start()

    def bd(W):
        Wp = jnp.pad(W, ((0, 0), (0, _LPAD - _L), (0, _LPAD - _L)))
        z = jnp.zeros_like(Wp)
        rows = [jnp.concatenate([Wp if c == r else z for c in range(4)], axis=2)
                for r in range(4)]
        return jnp.concatenate(rows, axis=1)

    WsB = bd(ws_ref[...])
    WnB = bd(wn_ref[...])
    bB = jnp.tile(jnp.pad(b_ref[...], ((0, 0), (0, _LPAD - _L))), (1, 32))

    pltpu.make_async_copy(adj_hbm, adj_ref, adj_sem).wait()
    A = adj_ref[...]
    Ab = (A != 0).astype(jnp.float32)
    deg = jnp.sum(Ab, axis=0)
    deg_inv = jnp.where(deg > 0, 1.0 / jnp.maximum(deg, 1.0), 0.0)
    A_s = Ab * deg_inv[None, :]

    dnT = (((0,), (0,)), ((), ()))
    dnW = (((1,), (1,)), ((), ()))

    def aggT(H):
        return jax.lax.dot_general(A_s, H, dnT,
                                   preferred_element_type=jnp.float32)

    def lin(H, W2):
        cols = [
            jax.lax.dot_general(H[:, 128 * t:128 * (t + 1)], W2, dnW,
                                preferred_element_type=jnp.float32)
            for t in range(H.shape[1] // 128)
        ]
        return jnp.concatenate(cols, axis=1)

    pltpu.make_async_copy(xn_hbm, xn_ref, xn_sem).wait()
    Xn = xn_ref[...].astype(jnp.float32)

    T = aggT(Xn)
    U = 4.0 * Xn
    V = deg[:, None] * T
    AV = aggT(V)
    U = lin(U, WsB[0]) + lin(4.0 * T, WnB[0]) + bB[0][None, :]
    V = lin(V, WsB[0]) + lin(AV, WnB[0]) + bB[0][None, :]
    AU = aggT(U)
    AV = aggT(V)
    U = lin(U, WsB[1]) + lin(AU, WnB[1]) + bB[1][None, :]
    V = lin(V, WsB[1]) + lin(AV, WnB[1]) + bB[1][None, :]

    AU = aggT(U)
    AV = aggT(V)
    U = lin(U, WsB[2]) + lin(AU, WnB[2]) + bB[2][None, :]
    u_s[...] = U.astype(jnp.bfloat16)
    pltpu.make_async_copy(u_s, u_hbm, out_sems.at[0]).start()
    half = pl.ds(0, 512)
    Vl = (lin(V[:, :512], WsB[2]) + lin(AV[:, :512], WnB[2])
          + bB[2][None, :512])
    v_s[:, :512] = Vl.astype(jnp.bfloat16)
    pltpu.make_async_copy(v_s.at[:, half], v_hbm.at[:, half],
                          out_sems.at[1]).start()
    half2 = pl.ds(512, 512)
    Vr = (lin(V[:, 512:], WsB[2]) + lin(AV[:, 512:], WnB[2])
          + bB[2][None, 512:])
    v_s[:, 512:] = Vr.astype(jnp.bfloat16)
    pltpu.make_async_copy(v_s.at[:, half2], v_hbm.at[:, half2],
                          out_sems.at[2]).start()
    pltpu.make_async_copy(u_s, u_hbm, out_sems.at[0]).wait()
    pltpu.make_async_copy(v_s.at[:, half], v_hbm.at[:, half],
                          out_sems.at[1]).wait()
    pltpu.make_async_copy(v_s.at[:, half2], v_hbm.at[:, half2],
                          out_sems.at[2]).wait()


def kernel(x, adj, W_self, b_self, W_neigh):
    nS, nC, nN, L = x.shape
    nG = nC * nS

    Xn = jnp.transpose(x, (2, 0, 1, 3))
    Xn = jnp.pad(Xn, ((0, 0), (0, 0), (0, 0), (0, _LPAD - L)))
    Xn = Xn.reshape(nN, nG * _LPAD).astype(jnp.bfloat16)

    U, V = pl.pallas_call(
        _gnn_body,
        in_specs=[
            pl.BlockSpec(memory_space=pl.ANY),
            pl.BlockSpec(memory_space=pl.ANY),
            pl.BlockSpec(memory_space=pltpu.VMEM),
            pl.BlockSpec(memory_space=pltpu.VMEM),
            pl.BlockSpec(memory_space=pltpu.VMEM),
        ],
        out_specs=[
            pl.BlockSpec(memory_space=pl.ANY),
            pl.BlockSpec(memory_space=pl.ANY),
        ],
        out_shape=[
            jax.ShapeDtypeStruct((nN, nG * _LPAD), jnp.bfloat16),
            jax.ShapeDtypeStruct((nN, nG * _LPAD), jnp.bfloat16),
        ],
        scratch_shapes=[
            pltpu.VMEM((nN, nG * _LPAD), jnp.bfloat16),
            pltpu.VMEM((nN, nN), jnp.float32),
            pltpu.VMEM((nN, nG * _LPAD), jnp.bfloat16),
            pltpu.VMEM((nN, nG * _LPAD), jnp.bfloat16),
            pltpu.SemaphoreType.DMA,
            pltpu.SemaphoreType.DMA,
            pltpu.SemaphoreType.DMA((3,)),
        ],
    )(Xn, adj, W_self, W_neigh, b_self)

    U = U.astype(jnp.float32)
    V = V.astype(jnp.float32)
    Ur = U.reshape(nN, nS, nC, _LPAD)[..., :L].transpose(1, 2, 0, 3)
    Vr = V.reshape(nN, nS, nC, _LPAD)[..., :L].transpose(1, 2, 0, 3)
    out = jnp.stack([Ur, Vr], axis=2).reshape(nS, 2 * nC, nN, L)
    return out
